# baseline (speedup 1.0000x reference)
import functools

import jax
import jax.numpy as jnp
from jax.experimental import pallas as pl
from jax.experimental.pallas import tpu as pltpu
from jax.experimental.pallas import tpu_sc as plsc

BATCH = 16384
RANK = 64
NUM_INDICES = 2 * BATCH
GATHER_WINDOW = 128


def _sc_gather(entity_embeddings, flat_indices):
  mesh = plsc.VectorSubcoreMesh(core_axis_name="core", subcore_axis_name="subcore")
  out_type = jax.ShapeDtypeStruct((NUM_INDICES, RANK), entity_embeddings.dtype)

  @functools.partial(
      pl.kernel, out_type=out_type, mesh=mesh,
      compiler_params=pltpu.CompilerParams(use_tc_tiling_on_sc=False))
  def gather_kernel(table_hbm, idx_hbm, out_hbm):
    def body(idx_vmem, out_vmem):
      pltpu.sync_copy(table_hbm.at[idx_vmem.at[0]], out_vmem)

    pltpu.emit_pipeline(
        body,
        grid=(NUM_INDICES // GATHER_WINDOW,),
        in_specs=[pl.BlockSpec((1, GATHER_WINDOW), lambda i: (0, i))],
        out_specs=[pl.BlockSpec((GATHER_WINDOW, RANK), lambda i: (i, 0))],
        core_axis_name=("core", "subcore"),
        dimension_semantics=(pltpu.PARALLEL,),
    )(idx_hbm, out_hbm)

  return gather_kernel(entity_embeddings, flat_indices.reshape(1, NUM_INDICES))


def _tc_classifier(rep, classifier_weight, classifier_bias, block_m=2048):
  num_relations = classifier_weight.shape[0]
  bias2d = classifier_bias.reshape(1, num_relations)

  def mm_kernel(rep_ref, w_ref, b_ref, o_ref):
    acc = jax.lax.dot_general(
        rep_ref[...], w_ref[...],
        dimension_numbers=(((1,), (1,)), ((), ())),
        preferred_element_type=jnp.float32,
    )
    o_ref[...] = acc + b_ref[...]

  return pl.pallas_call(
      mm_kernel,
      grid=(BATCH // block_m,),
      in_specs=[
          pl.BlockSpec((block_m, 2 * RANK), lambda i: (i, 0)),
          pl.BlockSpec((num_relations, 2 * RANK), lambda i: (0, 0)),
          pl.BlockSpec((1, num_relations), lambda i: (0, 0)),
      ],
      out_specs=pl.BlockSpec((block_m, num_relations), lambda i: (i, 0)),
      out_shape=jax.ShapeDtypeStruct((BATCH, num_relations), jnp.float32),
  )(rep, classifier_weight, bias2d)


def kernel(input_pairs, entity_embeddings, classifier_weight, classifier_bias):
  flat_indices = input_pairs.reshape(-1).astype(jnp.int32)
  gathered = _sc_gather(entity_embeddings, flat_indices)
  rep = gathered.reshape(BATCH, 2 * RANK)
  return _tc_classifier(rep, classifier_weight, classifier_bias)


# speedup vs baseline: 1.9191x; 1.9191x over previous
import functools

import jax
import jax.numpy as jnp
from jax.experimental import pallas as pl
from jax.experimental.pallas import tpu as pltpu
from jax.experimental.pallas import tpu_sc as plsc

ENTITY_SIZE = 1000000
BATCH = 16384
RANK = 64
NUM_IDX = 2 * BATCH
GATHER_WINDOW = 128
TBLOCK = 8192
HALF = TBLOCK // 2
NTBLK = (ENTITY_SIZE + TBLOCK - 1) // TBLOCK
PAIR_ROWS = NTBLK * HALF
MBLOCK = 2048


def _tc_transpose(table_t):
  eye = jnp.eye(RANK, dtype=jnp.float32)

  def tkernel(et_ref, eye_ref, out_ref):
    x = jax.lax.dot_general(
        et_ref[...], eye_ref[...],
        dimension_numbers=(((0,), (0,)), ((), ())),
        preferred_element_type=jnp.float32,
    )
    out_ref[...] = jnp.concatenate([x[:HALF, :], x[HALF:, :]], axis=1)

  return pl.pallas_call(
      tkernel,
      grid=(NTBLK,),
      in_specs=[
          pl.BlockSpec((RANK, TBLOCK), lambda i: (0, i)),
          pl.BlockSpec((RANK, RANK), lambda i: (0, 0)),
      ],
      out_specs=pl.BlockSpec((HALF, 2 * RANK), lambda i: (i, 0)),
      out_shape=jax.ShapeDtypeStruct((PAIR_ROWS, 2 * RANK), jnp.float32),
  )(table_t, eye)


def _sc_gather(pair_table, pair_indices):
  mesh = plsc.VectorSubcoreMesh(core_axis_name="core", subcore_axis_name="subcore")
  out_type = jax.ShapeDtypeStruct((NUM_IDX, 2 * RANK), jnp.float32)

  @functools.partial(
      pl.kernel, out_type=out_type, mesh=mesh,
      compiler_params=pltpu.CompilerParams(use_tc_tiling_on_sc=False))
  def gather_kernel(table_hbm, idx_hbm, out_hbm):
    def body(idx_vmem, out_vmem):
      pltpu.sync_copy(table_hbm.at[idx_vmem.at[0]], out_vmem)

    pltpu.emit_pipeline(
        body,
        grid=(NUM_IDX // GATHER_WINDOW,),
        in_specs=[pl.BlockSpec((1, GATHER_WINDOW), lambda i: (0, i))],
        out_specs=[pl.BlockSpec((GATHER_WINDOW, 2 * RANK), lambda i: (i, 0))],
        core_axis_name=("core", "subcore"),
        dimension_semantics=(pltpu.PARALLEL,),
    )(idx_hbm, out_hbm)

  return gather_kernel(pair_table, pair_indices)


def _tc_classifier(gathered, ps, po, w1d, w2d, bias2d):
  num_relations = bias2d.shape[1]

  def ckernel(g1_ref, g2_ref, ps_ref, po_ref, w1_ref, w2_ref, b_ref, o_ref):
    hi = jax.lax.broadcasted_iota(jnp.int32, (MBLOCK, 2 * RANK), 1) >= RANK
    g1 = jnp.where(hi == (ps_ref[...] == 1), g1_ref[...], 0.0)
    g2 = jnp.where(hi == (po_ref[...] == 1), g2_ref[...], 0.0)
    acc = jax.lax.dot_general(
        g1, w1_ref[...], dimension_numbers=(((1,), (0,)), ((), ())),
        preferred_element_type=jnp.float32)
    acc += jax.lax.dot_general(
        g2, w2_ref[...], dimension_numbers=(((1,), (0,)), ((), ())),
        preferred_element_type=jnp.float32)
    o_ref[...] = acc + b_ref[...]

  nblk = BATCH // MBLOCK
  return pl.pallas_call(
      ckernel,
      grid=(nblk,),
      in_specs=[
          pl.BlockSpec((MBLOCK, 2 * RANK), lambda i: (i, 0)),
          pl.BlockSpec((MBLOCK, 2 * RANK), lambda i, n=nblk: (i + n, 0)),
          pl.BlockSpec((MBLOCK, 1), lambda i: (i, 0)),
          pl.BlockSpec((MBLOCK, 1), lambda i: (i, 0)),
          pl.BlockSpec((2 * RANK, num_relations), lambda i: (0, 0)),
          pl.BlockSpec((2 * RANK, num_relations), lambda i: (0, 0)),
          pl.BlockSpec((1, num_relations), lambda i: (0, 0)),
      ],
      out_specs=pl.BlockSpec((MBLOCK, num_relations), lambda i: (i, 0)),
      out_shape=jax.ShapeDtypeStruct((BATCH, num_relations), jnp.float32),
  )(gathered, gathered, ps, po, w1d, w2d, bias2d)


def kernel(input_pairs, entity_embeddings, classifier_weight, classifier_bias):
  ip = input_pairs.astype(jnp.int32)
  flat = ip.T.reshape(1, NUM_IDX)
  pair_idx = (flat // TBLOCK) * HALF + flat % HALF
  half = (ip % TBLOCK) // HALF
  ps = half[:, 0:1]
  po = half[:, 1:2]

  w1d = jnp.concatenate([classifier_weight[:, :RANK].T,
                         classifier_weight[:, :RANK].T], axis=0)
  w2d = jnp.concatenate([classifier_weight[:, RANK:].T,
                         classifier_weight[:, RANK:].T], axis=0)
  bias2d = classifier_bias.reshape(1, -1)

  pair_table = _tc_transpose(entity_embeddings.T)
  gathered = _sc_gather(pair_table, pair_idx)
  return _tc_classifier(gathered, ps, po, w1d, w2d, bias2d)
